# Initial kernel scaffold; baseline (speedup 1.0000x reference)
import jax
import jax.numpy as jnp
from jax.experimental import pallas as pl


def kernel(dense_features, sparse_indices, emb_table, dW0, db0, dW1, db1, dW2, db2, oW0, ob0, oW1, ob1, oW2, ob2):
    raise NotImplementedError("write your pallas kernel here")


# speedup vs baseline: 3.7515x; 3.7515x over previous
import functools

import jax
import jax.numpy as jnp
import numpy as np
from jax import lax
from jax.experimental import pallas as pl
from jax.experimental.pallas import tpu as pltpu
from jax.experimental.pallas import tpu_sc as plsc

_B = 4096
_DENSE_IN = 13
_F = 26
_V = 1000000
_D = 32
_NFEAT = _F + 1
_NPAIR = (_NFEAT * (_NFEAT - 1)) // 2
_OVER_IN = _D + _NPAIR

_NC = 2
_NS = 16
_NW = _NC * _NS
_N_IDX = _B * _F
_B_PER_W = _N_IDX // _NW
_CHUNK = 128
_NCHUNK = _B_PER_W // _CHUNK

_BB = 512
_GRID = _B // _BB


def _build_perm() -> np.ndarray:
    iu0, iu1 = np.triu_indices(_NFEAT, k=1)
    ref_pos = {(int(i), int(k)): _D + p for p, (i, k) in enumerate(zip(iu0, iu1))}
    perm = list(range(_D))
    for delta in range(1, _NFEAT):
        for i in range(_NFEAT - delta):
            perm.append(ref_pos[(i, i + delta)])
    return np.array(perm, dtype=np.int32)


_PERM = _build_perm()

_S_NP = (np.arange(26 * _D)[:, None] // _D == np.arange(_F)[None, :]).astype(np.float32)


def _sc_gather(emb_table, idx3d):
    mesh = plsc.VectorSubcoreMesh(
        core_axis_name="c", subcore_axis_name="s",
        num_cores=_NC, num_subcores=_NS,
    )

    @functools.partial(
        pl.kernel,
        out_type=jax.ShapeDtypeStruct((_N_IDX, _D), jnp.float32),
        mesh=mesh,
        scratch_types=[
            pltpu.VMEM((_NCHUNK, _CHUNK), jnp.int32),
            pltpu.VMEM((_B_PER_W, _D), jnp.float32),
            pltpu.SemaphoreType.DMA,
        ],
        compiler_params=pltpu.CompilerParams(use_tc_tiling_on_sc=False),
    )
    def k(table_hbm, idx_hbm, out_hbm, idx_v, rows_v, sem):
        wid = lax.axis_index("s") * _NC + lax.axis_index("c")
        base = wid * _B_PER_W
        pltpu.sync_copy(idx_hbm.at[wid], idx_v)
        descs = []
        for j in range(_NCHUNK):
            descs.append(
                pltpu.async_copy(
                    table_hbm.at[idx_v.at[j]],
                    rows_v.at[pl.ds(j * _CHUNK, _CHUNK)],
                    sem,
                )
            )
        for d in descs:
            d.wait()
        pltpu.sync_copy(rows_v, out_hbm.at[pl.ds(base, _B_PER_W)])

    return k(emb_table, idx3d)


def _tc_body(x_ref, emb_ref, s_ref,
             dW0_ref, db0_ref, dW1_ref, db1_ref, dW2_ref, db2_ref,
             oW0_ref, ob0_ref, oW1_ref, ob1_ref, oW2_ref, ob2_ref,
             out_ref):
    f32 = jnp.float32
    hi = lax.Precision.HIGHEST

    def mm(a, b):
        return jnp.dot(a, b, precision=hi, preferred_element_type=f32)

    x = x_ref[...]
    h = jnp.maximum(mm(x, dW0_ref[...]) + db0_ref[...], 0.0)
    h = jnp.maximum(mm(h, dW1_ref[...]) + db1_ref[...], 0.0)
    d = jnp.maximum(mm(h, dW2_ref[...]) + db2_ref[...], 0.0)

    comb = jnp.concatenate([d, emb_ref[...]], axis=1)
    s = s_ref[...]
    pieces = [d]
    for delta in range(1, _NFEAT):
        w = (_NFEAT - delta) * _D
        prod = comb[:, :w] * comb[:, delta * _D: delta * _D + w]
        pieces.append(mm(prod, s[:w, :_NFEAT - delta]))
    cat = jnp.concatenate(pieces, axis=1)

    o = jnp.maximum(mm(cat, oW0_ref[...]) + ob0_ref[...], 0.0)
    o = jnp.maximum(mm(o, oW1_ref[...]) + ob1_ref[...], 0.0)
    out_ref[...] = mm(o, oW2_ref[...]) + ob2_ref[...]


def _tc_forward(dense, emb2d, s_mat, dW0, db0, dW1, db1, dW2, db2,
                oW0p, ob0, oW1, ob1, oW2p, ob2p):
    full = lambda a: pl.BlockSpec(a.shape, lambda i: (0,) * a.ndim)
    return pl.pallas_call(
        _tc_body,
        grid=(_GRID,),
        in_specs=[
            pl.BlockSpec((_BB, _DENSE_IN), lambda i: (i, 0)),
            pl.BlockSpec((_BB, _F * _D), lambda i: (i, 0)),
            full(s_mat),
            full(dW0), full(db0), full(dW1), full(db1), full(dW2), full(db2),
            full(oW0p), full(ob0), full(oW1), full(ob1), full(oW2p), full(ob2p),
        ],
        out_specs=pl.BlockSpec((_BB, 128), lambda i: (i, 0)),
        out_shape=jax.ShapeDtypeStruct((_B, 128), jnp.float32),
        compiler_params=pltpu.CompilerParams(
            dimension_semantics=("arbitrary",),
        ),
    )(dense, emb2d, s_mat, dW0, db0, dW1, db1, dW2, db2,
      oW0p, ob0, oW1, ob1, oW2p, ob2p)


def kernel(dense_features, sparse_indices, emb_table,
           dW0, db0, dW1, db1, dW2, db2,
           oW0, ob0, oW1, ob1, oW2, ob2):
    idx3d = sparse_indices.astype(jnp.int32).reshape(_NW, _NCHUNK, _CHUNK)
    gathered = _sc_gather(emb_table, idx3d)
    emb2d = gathered.reshape(_B, _F * _D)

    s_mat = jnp.asarray(_S_NP)
    oW0p = oW0[jnp.asarray(_PERM)]
    oW2p = jnp.pad(oW2, ((0, 0), (0, 127)))
    ob2p = jnp.pad(ob2.reshape(1, 1), ((0, 0), (0, 127)))
    out = _tc_forward(
        dense_features, emb2d, s_mat,
        dW0, db0.reshape(1, -1), dW1, db1.reshape(1, -1), dW2, db2.reshape(1, -1),
        oW0p, ob0.reshape(1, -1), oW1, ob1.reshape(1, -1), oW2p, ob2p,
    )
    return out[:, :1]


# speedup vs baseline: 4.9554x; 1.3209x over previous
import functools

import jax
import jax.numpy as jnp
import numpy as np
from jax import lax
from jax.experimental import pallas as pl
from jax.experimental.pallas import tpu as pltpu
from jax.experimental.pallas import tpu_sc as plsc

_B = 4096
_DENSE_IN = 13
_F = 26
_V = 1000000
_D = 32
_NFEAT = _F + 1
_NPAIR = (_NFEAT * (_NFEAT - 1)) // 2
_OVER_IN = _D + _NPAIR

_NC = 2
_NS = 16
_NW = _NC * _NS
_N_IDX = _B * _F
_B_PER_W = _N_IDX // _NW
_CHUNK = 128
_NCHUNK = _B_PER_W // _CHUNK

_BB = 1024
_GRID = _B // _BB


def _build_perm() -> np.ndarray:
    iu0, iu1 = np.triu_indices(_NFEAT, k=1)
    ref_pos = {(int(i), int(k)): _D + p for p, (i, k) in enumerate(zip(iu0, iu1))}
    perm = list(range(_D))
    for delta in range(1, _NFEAT):
        for i in range(_NFEAT - delta):
            perm.append(ref_pos[(i, i + delta)])
    return np.array(perm, dtype=np.int32)


_PERM = _build_perm()

_S_NP = (np.arange(26 * _D)[:, None] // _D == np.arange(_F)[None, :]).astype(np.float32)


def _sc_gather(emb_table, idx3d):
    mesh = plsc.VectorSubcoreMesh(
        core_axis_name="c", subcore_axis_name="s",
        num_cores=_NC, num_subcores=_NS,
    )

    @functools.partial(
        pl.kernel,
        out_type=jax.ShapeDtypeStruct((_N_IDX, _D), jnp.float32),
        mesh=mesh,
        scratch_types=[
            pltpu.VMEM((_NCHUNK, _CHUNK), jnp.int32),
            pltpu.VMEM((_B_PER_W, _D), jnp.float32),
            pltpu.SemaphoreType.DMA,
        ],
        compiler_params=pltpu.CompilerParams(use_tc_tiling_on_sc=False),
    )
    def k(table_hbm, idx_hbm, out_hbm, idx_v, rows_v, sem):
        wid = lax.axis_index("s") * _NC + lax.axis_index("c")
        base = wid * _B_PER_W
        pltpu.sync_copy(idx_hbm.at[wid], idx_v)
        descs = []
        for j in range(_NCHUNK):
            descs.append(
                pltpu.async_copy(
                    table_hbm.at[idx_v.at[j]],
                    rows_v.at[pl.ds(j * _CHUNK, _CHUNK)],
                    sem,
                )
            )
        for d in descs:
            d.wait()
        pltpu.sync_copy(rows_v, out_hbm.at[pl.ds(base, _B_PER_W)])

    return k(emb_table, idx3d)


def _tc_body(x_ref, emb_ref, s_ref,
             dW0_ref, db0_ref, dW1_ref, db1_ref, dW2_ref, db2_ref,
             oW0_ref, ob0_ref, oW1_ref, ob1_ref, oW2_ref, ob2_ref,
             out_ref):
    f32 = jnp.float32
    def mm(a, b):
        return jnp.dot(a, b, preferred_element_type=f32)

    x = x_ref[...]
    h = jnp.maximum(mm(x, dW0_ref[...]) + db0_ref[...], 0.0)
    h = jnp.maximum(mm(h, dW1_ref[...]) + db1_ref[...], 0.0)
    d = jnp.maximum(mm(h, dW2_ref[...]) + db2_ref[...], 0.0)

    comb = jnp.concatenate([d, emb_ref[...]], axis=1)
    shifted = [comb, comb[:, _D:], comb[:, 2 * _D:], comb[:, 3 * _D:]]
    s = s_ref[...]
    pieces = [d]
    for delta in range(1, _NFEAT):
        w = (_NFEAT - delta) * _D
        off = delta * _D
        src = shifted[delta % 4]
        base = (off // 128) * 128
        prod = comb[:, :w] * src[:, base: base + w]
        pieces.append(mm(prod, s[:w, :_NFEAT - delta]))
    cat = jnp.concatenate(pieces, axis=1)

    o = jnp.maximum(mm(cat, oW0_ref[...]) + ob0_ref[...], 0.0)
    o = jnp.maximum(mm(o, oW1_ref[...]) + ob1_ref[...], 0.0)
    out_ref[...] = mm(o, oW2_ref[...]) + ob2_ref[...]


def _tc_forward(dense, emb2d, s_mat, dW0, db0, dW1, db1, dW2, db2,
                oW0p, ob0, oW1, ob1, oW2p, ob2p):
    full = lambda a: pl.BlockSpec(a.shape, lambda i: (0,) * a.ndim)
    return pl.pallas_call(
        _tc_body,
        grid=(_GRID,),
        in_specs=[
            pl.BlockSpec((_BB, _DENSE_IN), lambda i: (i, 0)),
            pl.BlockSpec((_BB, _F * _D), lambda i: (i, 0)),
            full(s_mat),
            full(dW0), full(db0), full(dW1), full(db1), full(dW2), full(db2),
            full(oW0p), full(ob0), full(oW1), full(ob1), full(oW2p), full(ob2p),
        ],
        out_specs=pl.BlockSpec((_BB, 128), lambda i: (i, 0)),
        out_shape=jax.ShapeDtypeStruct((_B, 128), jnp.float32),
        compiler_params=pltpu.CompilerParams(
            dimension_semantics=("arbitrary",),
        ),
    )(dense, emb2d, s_mat, dW0, db0, dW1, db1, dW2, db2,
      oW0p, ob0, oW1, ob1, oW2p, ob2p)


def kernel(dense_features, sparse_indices, emb_table,
           dW0, db0, dW1, db1, dW2, db2,
           oW0, ob0, oW1, ob1, oW2, ob2):
    idx3d = sparse_indices.astype(jnp.int32).reshape(_NW, _NCHUNK, _CHUNK)
    gathered = _sc_gather(emb_table, idx3d)
    emb2d = gathered.reshape(_B, _F * _D)

    s_mat = jnp.asarray(_S_NP)
    oW0p = oW0[jnp.asarray(_PERM)]
    oW2p = jnp.pad(oW2, ((0, 0), (0, 127)))
    ob2p = jnp.pad(ob2.reshape(1, 1), ((0, 0), (0, 127)))
    out = _tc_forward(
        dense_features, emb2d, s_mat,
        dW0, db0.reshape(1, -1), dW1, db1.reshape(1, -1), dW2, db2.reshape(1, -1),
        oW0p, ob0.reshape(1, -1), oW1, ob1.reshape(1, -1), oW2p, ob2p,
    )
    return out[:, :1]
